# baseline (speedup 1.0000x reference)
import functools

import jax
import jax.numpy as jnp
from jax import lax
from jax.experimental import pallas as pl
from jax.experimental.pallas import tpu as pltpu
from jax.experimental.pallas import tpu_sc as plsc

_TW = 32768


def _repack_body(x0_ref, x1_ref, out_ref):
    x = jnp.concatenate([x0_ref[...], x1_ref[...]], axis=0)
    out_ref[...] = jnp.transpose(x, (1, 0))


def _repack(tablet):
    D, V = tablet.shape
    spans = (V + _TW - 1) // _TW
    h = _TW // 2
    c = 2048
    nq = h // c
    nb_last = (V - 1) // c

    return pl.pallas_call(
        _repack_body,
        grid=(spans * nq,),
        in_specs=[
            pl.BlockSpec(
                (D, c),
                lambda t: (0, jnp.minimum((t // nq) * 2 * nq + t % nq, nb_last)),
            ),
            pl.BlockSpec(
                (D, c),
                lambda t: (
                    0,
                    jnp.minimum((t // nq) * 2 * nq + nq + t % nq, nb_last),
                ),
            ),
        ],
        out_specs=pl.BlockSpec((c, 128), lambda t: (t, 0)),
        out_shape=jax.ShapeDtypeStruct((spans * h, 128), jnp.float32),
    )(tablet, tablet)


def _embed_call(B, D, b_per_w, num_cores):
    mesh = plsc.VectorSubcoreMesh(core_axis_name="c", subcore_axis_name="s")

    @functools.partial(
        pl.kernel,
        mesh=mesh,
        out_type=jax.ShapeDtypeStruct((B, D), jnp.float32),
        scratch_types=[
            pltpu.VMEM((b_per_w,), jnp.int32),
            pltpu.VMEM((b_per_w,), jnp.int32),
            pltpu.VMEM((b_per_w, 2 * D), jnp.float32),
            pltpu.VMEM((b_per_w, D), jnp.float32),
            pltpu.SemaphoreType.DMA,
        ],
        compiler_params=pltpu.CompilerParams(use_tc_tiling_on_sc=False),
    )
    def k(labels_hbm, table_hbm, out_hbm, lab_v, idx_v, rows_v, out_v, sem):
        wid = lax.axis_index("s") * num_cores + lax.axis_index("c")
        base = wid * b_per_w
        pltpu.sync_copy(labels_hbm.at[pl.ds(base, b_per_w)], lab_v)
        sh_half = _TW.bit_length() - 2
        for j in range(b_per_w // 16):
            lab = lab_v[pl.ds(j * 16, 16)]
            idx_v[pl.ds(j * 16, 16)] = lax.shift_left(
                lax.shift_right_logical(lab, sh_half + 1), sh_half
            ) + (lab & (_TW // 2 - 1))
        pltpu.async_copy(table_hbm.at[idx_v], rows_v, sem).wait()

        def body(g):
            lab16 = lab_v[pl.ds(g * 16, 16)]
            off16 = (lax.shift_right_logical(lab16, _TW.bit_length() - 2) & 1) * D
            for k in range(16):
                r = g * 16 + k
                off = off16[k]
                for c in range(D // 16):
                    out_v[r, pl.ds(c * 16, 16)] = rows_v[r, pl.ds(off + c * 16, 16)]

        pl.loop(0, b_per_w // 16)(body)
        pltpu.sync_copy(out_v, out_hbm.at[pl.ds(base, b_per_w)])

    return k


def kernel(labels, table):
    B = labels.shape[0]
    V, D = table.shape
    info = plsc.get_sparse_core_info()
    nw = info.num_cores * info.num_subcores
    b_per_w = B // nw
    labels = labels.astype(jnp.int32)
    table2 = _repack(table.T)
    return _embed_call(B, D, b_per_w, info.num_cores)(labels, table2)


# speedup vs baseline: 1.5470x; 1.5470x over previous
import functools

import jax
import jax.numpy as jnp
from jax import lax
from jax.experimental import pallas as pl
from jax.experimental.pallas import tpu as pltpu
from jax.experimental.pallas import tpu_sc as plsc

_TW = 32768


def _repack_body(xt_ref, out_ref):
    h = _TW // 2
    c = 2048
    for q in range(h // c):
        s = q * c
        x = jnp.concatenate(
            [xt_ref[:, pl.ds(s, c)], xt_ref[:, pl.ds(h + s, c)]], axis=0
        )
        out_ref[pl.ds(s, c), :] = jnp.transpose(x, (1, 0))


def _repack(tablet):
    D, V = tablet.shape
    spans = (V + _TW - 1) // _TW
    h = _TW // 2
    return pl.pallas_call(
        _repack_body,
        grid=(spans,),
        in_specs=[pl.BlockSpec((D, _TW), lambda g: (0, g))],
        out_specs=pl.BlockSpec((h, 128), lambda g: (g, 0)),
        out_shape=jax.ShapeDtypeStruct((spans * h, 128), jnp.float32),
    )(tablet)


def _embed_call(B, D, b_per_w, num_cores):
    mesh = plsc.VectorSubcoreMesh(core_axis_name="c", subcore_axis_name="s")

    @functools.partial(
        pl.kernel,
        mesh=mesh,
        out_type=jax.ShapeDtypeStruct((B, D), jnp.float32),
        scratch_types=[
            pltpu.VMEM((b_per_w,), jnp.int32),
            pltpu.VMEM((b_per_w,), jnp.int32),
            pltpu.VMEM((b_per_w, 2 * D), jnp.float32),
            pltpu.VMEM((b_per_w, D), jnp.float32),
            pltpu.SemaphoreType.DMA,
        ],
        compiler_params=pltpu.CompilerParams(use_tc_tiling_on_sc=False),
    )
    def k(labels_hbm, table_hbm, out_hbm, lab_v, idx_v, rows_v, out_v, sem):
        wid = lax.axis_index("s") * num_cores + lax.axis_index("c")
        base = wid * b_per_w
        pltpu.sync_copy(labels_hbm.at[pl.ds(base, b_per_w)], lab_v)
        sh_half = _TW.bit_length() - 2
        for j in range(b_per_w // 16):
            lab = lab_v[pl.ds(j * 16, 16)]
            idx_v[pl.ds(j * 16, 16)] = lax.shift_left(
                lax.shift_right_logical(lab, sh_half + 1), sh_half
            ) + (lab & (_TW // 2 - 1))
        pltpu.async_copy(table_hbm.at[idx_v], rows_v, sem).wait()

        def body(g):
            lab16 = lab_v[pl.ds(g * 16, 16)]
            off16 = (lax.shift_right_logical(lab16, _TW.bit_length() - 2) & 1) * D
            for k in range(16):
                r = g * 16 + k
                off = off16[k]
                for c in range(D // 16):
                    out_v[r, pl.ds(c * 16, 16)] = rows_v[r, pl.ds(off + c * 16, 16)]

        pl.loop(0, b_per_w // 16)(body)
        pltpu.sync_copy(out_v, out_hbm.at[pl.ds(base, b_per_w)])

    return k


def kernel(labels, table):
    B = labels.shape[0]
    V, D = table.shape
    info = plsc.get_sparse_core_info()
    nw = info.num_cores * info.num_subcores
    b_per_w = B // nw
    labels = labels.astype(jnp.int32)
    table2 = _repack(table.T)
    return _embed_call(B, D, b_per_w, info.num_cores)(labels, table2)


# speedup vs baseline: 1.5675x; 1.0132x over previous
import functools

import jax
import jax.numpy as jnp
from jax import lax
from jax.experimental import pallas as pl
from jax.experimental.pallas import tpu as pltpu
from jax.experimental.pallas import tpu_sc as plsc

_TW = 32768


def _repack_body(xt_ref, out_ref):
    h = _TW // 2
    c = 2048
    for q in range(h // c):
        s = q * c
        x = jnp.concatenate(
            [xt_ref[:, pl.ds(s, c)], xt_ref[:, pl.ds(h + s, c)]], axis=0
        )
        out_ref[pl.ds(s, c), :] = jnp.transpose(x, (1, 0))


def _repack(tablet):
    D, V = tablet.shape
    spans = (V + _TW - 1) // _TW
    h = _TW // 2
    return pl.pallas_call(
        _repack_body,
        grid=(spans,),
        in_specs=[pl.BlockSpec((D, _TW), lambda g: (0, g))],
        out_specs=pl.BlockSpec((h, 128), lambda g: (g, 0)),
        out_shape=jax.ShapeDtypeStruct((spans * h, 128), jnp.float32),
    )(tablet)


def _embed_call(B, D, b_per_w, num_cores):
    mesh = plsc.VectorSubcoreMesh(core_axis_name="c", subcore_axis_name="s")

    @functools.partial(
        pl.kernel,
        mesh=mesh,
        out_type=jax.ShapeDtypeStruct((B, D), jnp.float32),
        scratch_types=[
            pltpu.VMEM((b_per_w,), jnp.int32),
            pltpu.VMEM((b_per_w,), jnp.int32),
            pltpu.VMEM((b_per_w, 2 * D), jnp.float32),
            pltpu.VMEM((b_per_w, D), jnp.float32),
            pltpu.SemaphoreType.DMA,
            pltpu.SemaphoreType.DMA,
        ],
        compiler_params=pltpu.CompilerParams(use_tc_tiling_on_sc=False),
    )
    def k(labels_hbm, table_hbm, out_hbm, lab_v, idx_v, rows_v, out_v, gsem, osem):
        wid = lax.axis_index("s") * num_cores + lax.axis_index("c")
        base = wid * b_per_w
        nch = 4
        ch = b_per_w // nch
        pltpu.sync_copy(labels_hbm.at[pl.ds(base, b_per_w)], lab_v)
        sh_half = _TW.bit_length() - 2
        for j in range(b_per_w // 16):
            lab = lab_v[pl.ds(j * 16, 16)]
            idx_v[pl.ds(j * 16, 16)] = lax.shift_left(
                lax.shift_right_logical(lab, sh_half + 1), sh_half
            ) + (lab & (_TW // 2 - 1))

        def gather(i):
            return pltpu.async_copy(
                table_hbm.at[idx_v.at[pl.ds(i * ch, ch)]],
                rows_v.at[pl.ds(i * ch, ch)],
                gsem,
            )

        def select(i):
            def body(g):
                lab16 = lab_v[pl.ds(g * 16, 16)]
                off16 = (lax.shift_right_logical(lab16, sh_half) & 1) * D
                for k in range(16):
                    r = g * 16 + k
                    off = off16[k]
                    for c in range(D // 16):
                        out_v[r, pl.ds(c * 16, 16)] = rows_v[
                            r, pl.ds(off + c * 16, 16)
                        ]

            pl.loop(i * (ch // 16), (i + 1) * (ch // 16))(body)

        handles = [gather(0)]
        outs = []
        for i in range(nch):
            if i + 1 < nch:
                handles.append(gather(i + 1))
            handles[i].wait()
            select(i)
            outs.append(
                pltpu.async_copy(
                    out_v.at[pl.ds(i * ch, ch)],
                    out_hbm.at[pl.ds(base + i * ch, ch)],
                    osem,
                )
            )
        for o in outs:
            o.wait()

    return k


def kernel(labels, table):
    B = labels.shape[0]
    V, D = table.shape
    info = plsc.get_sparse_core_info()
    nw = info.num_cores * info.num_subcores
    b_per_w = B // nw
    labels = labels.astype(jnp.int32)
    table2 = _repack(table.T)
    return _embed_call(B, D, b_per_w, info.num_cores)(labels, table2)
